# Initial kernel scaffold; baseline (speedup 1.0000x reference)
import jax
import jax.numpy as jnp
from jax.experimental import pallas as pl


def kernel(inputs, labels, sample_ids, weight):
    raise NotImplementedError("write your pallas kernel here")


# speedup vs baseline: 7.2193x; 7.2193x over previous
import functools

import jax
import jax.numpy as jnp
from jax import lax
from jax.experimental import pallas as pl
from jax.experimental.pallas import tpu as pltpu
from jax.experimental.pallas import tpu_sc as plsc

NTOKENS = 100000
NHID = 128
NSAMPLED = 1024
BATCH = 1024
NROWS = BATCH + NSAMPLED


def _sc_gather(weight, idx):
    info = plsc.get_sparse_core_info()
    nc, ns = info.num_cores, info.num_subcores
    nw = nc * ns
    b_per_w = NROWS // nw

    mesh = plsc.VectorSubcoreMesh(core_axis_name="c", subcore_axis_name="s")

    @functools.partial(
        pl.kernel,
        mesh=mesh,
        out_type=jax.ShapeDtypeStruct((NROWS, NHID), jnp.float32),
        scratch_types=[
            pltpu.VMEM((b_per_w,), jnp.int32),
            pltpu.VMEM((b_per_w, NHID), jnp.float32),
            pltpu.SemaphoreType.DMA,
        ],
    )
    def gather_kernel(table_hbm, idx_hbm, out_hbm, idx_v, rows_v, sem):
        wid = lax.axis_index("s") * nc + lax.axis_index("c")
        base = wid * b_per_w
        pltpu.sync_copy(idx_hbm.at[pl.ds(base, b_per_w)], idx_v)
        pltpu.async_copy(table_hbm.at[idx_v], rows_v, sem).wait()
        pltpu.sync_copy(rows_v, out_hbm.at[pl.ds(base, b_per_w)])

    return gather_kernel(weight, idx)


def _tc_body(x_ref, rows_ref, out_ref):
    x = x_ref[...]
    tw = rows_ref[0:BATCH, :]
    sw = rows_ref[BATCH:NROWS, :]

    d = x - tw
    true_norm = jnp.sqrt(jnp.sum(d * d, axis=1))

    xn = jnp.sum(x * x, axis=1)
    sn = jnp.sum(sw * sw, axis=1)
    g = lax.dot_general(x, sw, (((1,), (1,)), ((), ())),
                        preferred_element_type=jnp.float32)
    dist2 = xn[:, None] + sn[None, :] - 2.0 * g
    dist = jnp.sqrt(jnp.maximum(dist2, 0.0))
    s = jnp.sum(jnp.exp(dist), axis=1)
    out_ref[...] = true_norm - jnp.log(s)


def kernel(inputs, labels, sample_ids, weight):
    idx = jnp.concatenate(
        [labels.astype(jnp.int32), sample_ids.astype(jnp.int32)])
    rows = _sc_gather(weight, idx)

    return pl.pallas_call(
        _tc_body,
        out_shape=jax.ShapeDtypeStruct((BATCH,), jnp.float32),
    )(inputs, rows)


# speedup vs baseline: 7.4929x; 1.0379x over previous
import functools

import jax
import jax.numpy as jnp
from jax import lax
from jax.experimental import pallas as pl
from jax.experimental.pallas import tpu as pltpu
from jax.experimental.pallas import tpu_sc as plsc

NTOKENS = 100000
NHID = 128
NSAMPLED = 1024
BATCH = 1024
NROWS = BATCH + NSAMPLED


def _sc_gather(weight, idx):
    info = plsc.get_sparse_core_info()
    nc, ns = 1, info.num_subcores
    nw = nc * ns
    b_per_w = NROWS // nw

    mesh = plsc.VectorSubcoreMesh(core_axis_name="c", subcore_axis_name="s",
                                  num_cores=nc)

    @functools.partial(
        pl.kernel,
        mesh=mesh,
        out_type=jax.ShapeDtypeStruct((NROWS, NHID), jnp.float32),
        scratch_types=[
            pltpu.VMEM((b_per_w,), jnp.int32),
            pltpu.VMEM((b_per_w, NHID), jnp.float32),
            pltpu.SemaphoreType.DMA,
        ],
    )
    def gather_kernel(table_hbm, idx_hbm, out_hbm, idx_v, rows_v, sem):
        wid = lax.axis_index("s") * nc + lax.axis_index("c")
        base = wid * b_per_w
        pltpu.sync_copy(idx_hbm.at[pl.ds(base, b_per_w)], idx_v)
        pltpu.async_copy(table_hbm.at[idx_v], rows_v, sem).wait()
        pltpu.sync_copy(rows_v, out_hbm.at[pl.ds(base, b_per_w)])

    return gather_kernel(weight, idx)


def _tc_body(x_ref, rows_ref, out_ref):
    x = x_ref[...]
    tw = rows_ref[0:BATCH, :]
    sw = rows_ref[BATCH:NROWS, :]

    d = x - tw
    true_norm = jnp.sqrt(jnp.sum(d * d, axis=1))

    xn = jnp.sum(x * x, axis=1)
    sn = jnp.sum(sw * sw, axis=1)
    g = lax.dot_general(x, sw, (((1,), (1,)), ((), ())),
                        preferred_element_type=jnp.float32)
    dist2 = xn[:, None] + sn[None, :] - 2.0 * g
    dist = jnp.sqrt(jnp.maximum(dist2, 0.0))
    s = jnp.sum(jnp.exp(dist), axis=1)
    out_ref[...] = true_norm - jnp.log(s)


def kernel(inputs, labels, sample_ids, weight):
    idx = jnp.concatenate(
        [labels.astype(jnp.int32), sample_ids.astype(jnp.int32)])
    rows = _sc_gather(weight, idx)

    return pl.pallas_call(
        _tc_body,
        out_shape=jax.ShapeDtypeStruct((BATCH,), jnp.float32),
    )(inputs, rows)
